# baseline (speedup 1.0000x reference)
import jax
import jax.numpy as jnp
from jax import lax
from jax.experimental import pallas as pl
from jax.experimental.pallas import tpu as pltpu
from jax.experimental.pallas import tpu_sc as plsc

E = 64
D = 768
DFF = 3072
T = 2048
C = 40
NSLOT = (E + 1) * C
TRASH = E * C

TB = 256
FT = 3072

NC = 2
NS = 16
NW = NC * NS
TPW = T // NW


def _moe_body(xf_ref, wg_ref, w1_ref, b1_ref, w2_ref, b2_ref,
              out_ref, dst_out_ref, dst_s, scale_s):
    e = pl.program_id(0)

    @pl.when(e == 0)
    def _():
        xv = xf_ref[...]
        logits = jnp.dot(xv, wg_ref[...],
                         preferred_element_type=jnp.float32)
        m = jnp.max(logits, axis=1, keepdims=True)
        ex = jnp.exp(logits - m)
        s = jnp.sum(ex, axis=1, keepdims=True)
        probs = ex / s
        gate = jnp.max(probs, axis=1, keepdims=True)
        eids = lax.broadcasted_iota(jnp.int32, (T, E), 1)
        eidx = jnp.min(jnp.where(probs == gate, eids, E), axis=1,
                       keepdims=True)
        own = eids == eidx

        r = lax.broadcasted_iota(jnp.int32, (TB, TB), 0)
        c = lax.broadcasted_iota(jnp.int32, (TB, TB), 1)
        tril = (r >= c).astype(jnp.bfloat16)
        onehot = own.astype(jnp.bfloat16)
        carry = jnp.zeros((1, E), jnp.float32)
        pos_blocks = []
        for j in range(T // TB):
            oj = onehot[j * TB:(j + 1) * TB, :]
            cj = jnp.dot(tril, oj, preferred_element_type=jnp.float32) + carry
            carry = cj[TB - 1:TB, :]
            ownj = own[j * TB:(j + 1) * TB, :]
            pos_blocks.append(
                jnp.sum(jnp.where(ownj, cj, 0.0), axis=1, keepdims=True) - 1.0)
        posf = jnp.concatenate(pos_blocks, axis=0)
        pos = posf.astype(jnp.int32)
        keep = pos < C
        slot = eidx * C + jnp.minimum(pos, C - 1)
        dst_col = jnp.where(keep, slot, TRASH)
        dst_s[...] = dst_col
        scale_s[...] = jnp.where(keep, gate, 0.0)
        dst_out_ref[...] = dst_col

    @pl.when(e < E)
    def _():
        dst_col = dst_s[...]
        slot_row = e * C + lax.broadcasted_iota(jnp.int32, (1, C), 1)
        ohsT = (dst_col == slot_row).astype(jnp.float32)
        tdims = (((0,), (0,)), ((), ()))
        xb = lax.dot_general(ohsT, xf_ref[...], tdims,
                             preferred_element_type=jnp.float32)
        ssc = lax.dot_general(ohsT, scale_s[...], tdims,
                              preferred_element_type=jnp.float32)
        h = jnp.dot(xb, w1_ref[0], preferred_element_type=jnp.float32)
        h = jax.nn.gelu(h + b1_ref[0])
        part = jnp.dot(h, w2_ref[0], preferred_element_type=jnp.float32)
        out_ref[0] = (part + b2_ref[0]) * ssc

    @pl.when(e == E)
    def _():
        out_ref[0] = jnp.zeros((C, D), jnp.float32)


def _moe(xf, wg, w1, b1, w2, b2):
    def we(e):
        return jnp.minimum(e, E - 1)

    return pl.pallas_call(
        _moe_body,
        grid=(E + 1,),
        in_specs=[
            pl.BlockSpec((T, D), lambda e: (0, 0)),
            pl.BlockSpec((D, E), lambda e: (0, 0)),
            pl.BlockSpec((1, D, FT), lambda e: (we(e), 0, 0)),
            pl.BlockSpec((1, 1, FT), lambda e: (we(e), 0, 0)),
            pl.BlockSpec((1, FT, D), lambda e: (we(e), 0, 0)),
            pl.BlockSpec((1, 1, D), lambda e: (we(e), 0, 0)),
        ],
        out_specs=[
            pl.BlockSpec((1, C, D), lambda e: (e, 0, 0)),
            pl.BlockSpec((T, 1), lambda e: (0, 0)),
        ],
        out_shape=[
            jax.ShapeDtypeStruct((E + 1, C, D), jnp.float32),
            jax.ShapeDtypeStruct((T, 1), jnp.int32),
        ],
        scratch_shapes=[
            pltpu.VMEM((T, 1), jnp.int32),
            pltpu.VMEM((T, 1), jnp.float32),
        ],
        compiler_params=pltpu.CompilerParams(
            dimension_semantics=("arbitrary",)),
    )(xf, wg, w1, b1.reshape(E, 1, DFF), w2, b2.reshape(E, 1, D))


def _combine_body(outb_hbm, dst_hbm, y_hbm, idx_v, rows_v, sem):
    wid = lax.axis_index("s") * NC + lax.axis_index("c")
    base = wid * TPW
    pltpu.sync_copy(dst_hbm.at[pl.ds(base, TPW)], idx_v)
    pltpu.async_copy(outb_hbm.at[idx_v], rows_v, sem).wait()
    pltpu.sync_copy(rows_v, y_hbm.at[pl.ds(base, TPW)])


def _combine(outb, dst):
    mesh = plsc.VectorSubcoreMesh(
        core_axis_name="c", subcore_axis_name="s",
        num_cores=NC, num_subcores=NS)
    fn = pl.kernel(
        _combine_body,
        out_type=jax.ShapeDtypeStruct((T, D), jnp.float32),
        mesh=mesh,
        scratch_types=[
            pltpu.VMEM((TPW,), jnp.int32),
            pltpu.VMEM((TPW, D), jnp.float32),
            pltpu.SemaphoreType.DMA,
        ],
    )
    return fn(outb, dst)


def kernel(x, Wg, W1, b1, W2, b2):
    xf = x.reshape(T, D)
    outb, dst2 = _moe(xf, Wg, W1, b1, W2, b2)
    y = _combine(outb.reshape(NSLOT, D), dst2.reshape(T))
    return y.reshape(x.shape)


# speedup vs baseline: 1.0231x; 1.0231x over previous
import jax
import jax.numpy as jnp
from jax import lax
from jax.experimental import pallas as pl
from jax.experimental.pallas import tpu as pltpu
from jax.experimental.pallas import tpu_sc as plsc

E = 64
D = 768
DFF = 3072
T = 2048
C = 40
NSLOT = (E + 1) * C
TRASH = E * C

TB = 256
FT = 3072

NC = 2
NS = 16
NW = NC * NS
TPW = T // NW


def _moe_body(xf_ref, wg_ref, w1_ref, b1_ref, w2_ref, b2_ref,
              out_ref, dst_out_ref, dst_s, scale_s):
    e = pl.program_id(0)

    @pl.when(e == 0)
    def _():
        xv = xf_ref[...]
        logits = jnp.dot(xv, wg_ref[...],
                         preferred_element_type=jnp.float32)
        m = jnp.max(logits, axis=1, keepdims=True)
        ex = jnp.exp(logits - m)
        s = jnp.sum(ex, axis=1, keepdims=True)
        probs = ex / s
        gate = jnp.max(probs, axis=1, keepdims=True)
        eids = lax.broadcasted_iota(jnp.int32, (T, E), 1)
        eidx = jnp.min(jnp.where(probs == gate, eids, E), axis=1,
                       keepdims=True)
        own = eids == eidx

        r = lax.broadcasted_iota(jnp.int32, (TB, TB), 0)
        c = lax.broadcasted_iota(jnp.int32, (TB, TB), 1)
        tril = (r >= c).astype(jnp.bfloat16)
        onehot = own.astype(jnp.bfloat16)
        carry = jnp.zeros((1, E), jnp.float32)
        pos_blocks = []
        for j in range(T // TB):
            oj = onehot[j * TB:(j + 1) * TB, :]
            cj = jnp.dot(tril, oj, preferred_element_type=jnp.float32) + carry
            carry = cj[TB - 1:TB, :]
            ownj = own[j * TB:(j + 1) * TB, :]
            pos_blocks.append(
                jnp.sum(jnp.where(ownj, cj, 0.0), axis=1, keepdims=True) - 1.0)
        posf = jnp.concatenate(pos_blocks, axis=0)
        pos = posf.astype(jnp.int32)
        keep = pos < C
        slot = eidx * C + jnp.minimum(pos, C - 1)
        dst_col = jnp.where(keep, slot, TRASH)
        dst_s[...] = dst_col
        scale_s[...] = jnp.where(keep, gate, 0.0)
        dst_out_ref[...] = dst_col

    @pl.when(e < E)
    def _():
        dst_col = dst_s[...]
        slot_row = e * C + lax.broadcasted_iota(jnp.int32, (1, C), 1)
        ohsT = (dst_col == slot_row).astype(jnp.float32)
        tdims = (((0,), (0,)), ((), ()))
        xb = lax.dot_general(ohsT, xf_ref[...], tdims,
                             preferred_element_type=jnp.float32)
        ssc = lax.dot_general(ohsT, scale_s[...], tdims,
                              preferred_element_type=jnp.float32)
        h = jnp.dot(xb, w1_ref[0], preferred_element_type=jnp.float32)
        h = jax.nn.gelu(h + b1_ref[pl.ds(e, 1), :])
        part = jnp.dot(h, w2_ref[0], preferred_element_type=jnp.float32)
        out_ref[0] = (part + b2_ref[pl.ds(e, 1), :]) * ssc

    @pl.when(e == E)
    def _():
        out_ref[0] = jnp.zeros((C, D), jnp.float32)


def _moe(xf, wg, w1, b1, w2, b2):
    def we(e):
        return jnp.minimum(e, E - 1)

    return pl.pallas_call(
        _moe_body,
        grid=(E + 1,),
        in_specs=[
            pl.BlockSpec((T, D), lambda e: (0, 0)),
            pl.BlockSpec((D, E), lambda e: (0, 0)),
            pl.BlockSpec((1, D, FT), lambda e: (we(e), 0, 0)),
            pl.BlockSpec((E, DFF), lambda e: (0, 0)),
            pl.BlockSpec((1, FT, D), lambda e: (we(e), 0, 0)),
            pl.BlockSpec((E, D), lambda e: (0, 0)),
        ],
        out_specs=[
            pl.BlockSpec((1, C, D), lambda e: (e, 0, 0)),
            pl.BlockSpec((T, 1), lambda e: (0, 0)),
        ],
        out_shape=[
            jax.ShapeDtypeStruct((E + 1, C, D), jnp.float32),
            jax.ShapeDtypeStruct((T, 1), jnp.int32),
        ],
        scratch_shapes=[
            pltpu.VMEM((T, 1), jnp.int32),
            pltpu.VMEM((T, 1), jnp.float32),
        ],
        compiler_params=pltpu.CompilerParams(
            dimension_semantics=("arbitrary",)),
    )(xf, wg, w1, b1, w2, b2)


def _combine_body(outb_hbm, dst_hbm, y_hbm, idx_v, rows_v, sem):
    wid = lax.axis_index("s") * NC + lax.axis_index("c")
    base = wid * TPW
    pltpu.sync_copy(dst_hbm.at[pl.ds(base, TPW)], idx_v)
    pltpu.async_copy(outb_hbm.at[idx_v], rows_v, sem).wait()
    pltpu.sync_copy(rows_v, y_hbm.at[pl.ds(base, TPW)])


def _combine(outb, dst):
    mesh = plsc.VectorSubcoreMesh(
        core_axis_name="c", subcore_axis_name="s",
        num_cores=NC, num_subcores=NS)
    fn = pl.kernel(
        _combine_body,
        out_type=jax.ShapeDtypeStruct((T, D), jnp.float32),
        mesh=mesh,
        scratch_types=[
            pltpu.VMEM((TPW,), jnp.int32),
            pltpu.VMEM((TPW, D), jnp.float32),
            pltpu.SemaphoreType.DMA,
        ],
    )
    return fn(outb, dst)


def kernel(x, Wg, W1, b1, W2, b2):
    xf = x.reshape(T, D)
    outb, dst2 = _moe(xf, Wg, W1, b1, W2, b2)
    y = _combine(outb.reshape(NSLOT, D), dst2.reshape(T))
    return y.reshape(x.shape)
